# Initial kernel scaffold; baseline (speedup 1.0000x reference)
import jax
import jax.numpy as jnp
from jax.experimental import pallas as pl


def kernel(x, fps_morgan, fps_maccs, fps_erg, params, edge_index, graph_ids):
    raise NotImplementedError("write your pallas kernel here")


# speedup vs baseline: 3.1133x; 3.1133x over previous
import functools

import numpy as np

import jax
import jax.numpy as jnp
from jax import lax
from jax.experimental import pallas as pl
from jax.experimental.pallas import tpu as pltpu
from jax.experimental.pallas import tpu_sc as plsc

_N, _E, _D, _H, _B = 10000, 320000, 128, 128, 1024
_NC, _NS = 2, 16
_NW = _NC * _NS
_K = 80
_EC = _E // _NW
_C = _EC // _K
_NPAD = 10240
_CP = _NPAD // (_NW * _K)
_BPAD = 1152
_RT = 624
_SA = 2.0**12
_SB = 2.0**30


def _ln(z, g, b, eps=1e-5):
    m = jnp.mean(z, axis=-1, keepdims=True)
    v = jnp.mean((z - m) ** 2, axis=-1, keepdims=True)
    return (z - m) / jnp.sqrt(v + eps) * g + b


def _gelu(z):
    sqrt_half = np.sqrt(0.5).astype(np.float32)
    return 0.5 * z * (1.0 + lax.erf(z * sqrt_half))


def _seg_sum_sc(h, src_w, dst_w, zeros_n):
    mesh = plsc.VectorSubcoreMesh(core_axis_name="c", subcore_axis_name="s")
    tail = _N - _RT * _NS

    @functools.partial(
        pl.kernel,
        out_type=jax.ShapeDtypeStruct((2 * _N, _D), jnp.int32),
        mesh=mesh,
        scratch_types=[
            pltpu.VMEM((_C, _K), jnp.int32),
            pltpu.VMEM((_C, _K), jnp.int32),
            pltpu.VMEM((_K, _D), jnp.int32),
            pltpu.VMEM_SHARED((_N, _D), jnp.int32),
            pltpu.SemaphoreType.DMA,
        ],
    )
    def seg(h_hbm, src_hbm, dst_hbm, z_hbm, out_hbm,
            src_v, dst_v, rows_v, acc_sh, sem):
        cid = lax.axis_index("c")
        sid = lax.axis_index("s")
        wid = sid * _NC + cid
        pltpu.sync_copy(z_hbm.at[pl.ds(sid * _RT, _RT)],
                        acc_sh.at[pl.ds(sid * _RT, _RT)])

        @pl.when(sid == _NS - 1)
        def _zero_tail():
            pltpu.sync_copy(z_hbm.at[pl.ds(_RT * _NS, tail)],
                            acc_sh.at[pl.ds(_RT * _NS, tail)])

        pltpu.sync_copy(src_hbm.at[wid], src_v)
        pltpu.sync_copy(dst_hbm.at[wid], dst_v)
        plsc.subcore_barrier()

        def body(c, carry):
            pltpu.async_copy(h_hbm.at[src_v.at[c]], rows_v, sem).wait()
            pltpu.sync_copy(rows_v, acc_sh.at[dst_v.at[c]], add=True)
            return carry

        lax.fori_loop(0, _C, body, 0)
        plsc.subcore_barrier()
        pltpu.sync_copy(acc_sh.at[pl.ds(sid * _RT, _RT)],
                        out_hbm.at[pl.ds(cid * _N + sid * _RT, _RT)])

        @pl.when(sid == _NS - 1)
        def _write_tail():
            pltpu.sync_copy(acc_sh.at[pl.ds(_RT * _NS, tail)],
                            out_hbm.at[pl.ds(cid * _N + _RT * _NS, tail)])

    return seg(h, src_w, dst_w, zeros_n)


def _pool_sc(h, idx_w, gid_w, zeros_p, zeros_c, ones_k):
    mesh = plsc.VectorSubcoreMesh(core_axis_name="c", subcore_axis_name="s")
    zrt = _BPAD // _NS
    grt = _B // _NS

    @functools.partial(
        pl.kernel,
        out_type=(jax.ShapeDtypeStruct((2 * _B, _D), jnp.int32),
                  jax.ShapeDtypeStruct((2 * _B, _D), jnp.int32)),
        mesh=mesh,
        scratch_types=[
            pltpu.VMEM((_CP, _K), jnp.int32),
            pltpu.VMEM((_CP, _K), jnp.int32),
            pltpu.VMEM((_K, _D), jnp.int32),
            pltpu.VMEM((_K, _D), jnp.int32),
            pltpu.VMEM_SHARED((_BPAD, _D), jnp.int32),
            pltpu.VMEM_SHARED((_BPAD, _D), jnp.int32),
            pltpu.SemaphoreType.DMA,
        ],
    )
    def pool(h_hbm, idx_hbm, gid_hbm, zp_hbm, zc_hbm, ones_hbm,
             sum_hbm, cnt_hbm,
             idx_v, gid_v, rows_v, ones_v, acc_sh, cacc_sh, sem):
        cid = lax.axis_index("c")
        sid = lax.axis_index("s")
        wid = sid * _NC + cid
        pltpu.sync_copy(zp_hbm.at[pl.ds(sid * zrt, zrt)],
                        acc_sh.at[pl.ds(sid * zrt, zrt)])
        pltpu.sync_copy(zc_hbm.at[pl.ds(sid * zrt, zrt)],
                        cacc_sh.at[pl.ds(sid * zrt, zrt)])
        pltpu.sync_copy(ones_hbm, ones_v)
        pltpu.sync_copy(idx_hbm.at[wid], idx_v)
        pltpu.sync_copy(gid_hbm.at[wid], gid_v)
        plsc.subcore_barrier()

        def body(c, carry):
            pltpu.async_copy(h_hbm.at[idx_v.at[c]], rows_v, sem).wait()
            pltpu.sync_copy(rows_v, acc_sh.at[gid_v.at[c]], add=True)
            pltpu.sync_copy(ones_v, cacc_sh.at[gid_v.at[c]], add=True)
            return carry

        lax.fori_loop(0, _CP, body, 0)
        plsc.subcore_barrier()
        pltpu.sync_copy(acc_sh.at[pl.ds(sid * grt, grt)],
                        sum_hbm.at[pl.ds(cid * _B + sid * grt, grt)])
        pltpu.sync_copy(cacc_sh.at[pl.ds(sid * grt, grt)],
                        cnt_hbm.at[pl.ds(cid * _B + sid * grt, grt)])

    return pool(h, idx_w, gid_w, zeros_p, zeros_c, ones_k)


_ROWS = 1000


def _quantize(y, qa_ref, qb_ref):
    qa = jnp.round(y * _SA)
    qa_ref[...] = qa.astype(jnp.int32)
    resid = y - qa * (1.0 / _SA)
    qb_ref[...] = jnp.round(resid * _SB).astype(jnp.int32)


def _ln_body(x_ref, g_ref, b_ref, o_ref, qa_ref, qb_ref):
    y = _ln(x_ref[...], g_ref[...], b_ref[...])
    o_ref[...] = y
    _quantize(y, qa_ref, qb_ref)


def _ln_pallas(x, g, b):
    nb = _N // _ROWS
    return pl.pallas_call(
        _ln_body,
        grid=(nb,),
        in_specs=[
            pl.BlockSpec((_ROWS, _D), lambda i: (i, 0)),
            pl.BlockSpec((1, _D), lambda i: (0, 0)),
            pl.BlockSpec((1, _D), lambda i: (0, 0)),
        ],
        out_specs=(pl.BlockSpec((_ROWS, _D), lambda i: (i, 0)),
                   pl.BlockSpec((_ROWS, _D), lambda i: (i, 0)),
                   pl.BlockSpec((_ROWS, _D), lambda i: (i, 0))),
        out_shape=(jax.ShapeDtypeStruct((_N, _D), jnp.float32),
                   jax.ShapeDtypeStruct((_N, _D), jnp.int32),
                   jax.ShapeDtypeStruct((_N, _D), jnp.int32)),
    )(x, g.reshape(1, _D), b.reshape(1, _D))


def _gin_body(h_ref, a0_ref, a1_ref, b0_ref, b1_ref, w1_ref, c1_ref, g1_ref, e1_ref,
              w2_ref, c2_ref, g2_ref, e2_ref, ng_ref, nb_ref,
              o_ref, qa_ref, qb_ref, *, residual):
    h = h_ref[...]
    agg = ((a0_ref[...].astype(jnp.float32)
            + a1_ref[...].astype(jnp.float32)) * (1.0 / _SA)
           + (b0_ref[...].astype(jnp.float32)
              + b1_ref[...].astype(jnp.float32)) * (1.0 / _SB))
    z = h + agg
    z = jnp.dot(z, w1_ref[...], preferred_element_type=jnp.float32) + c1_ref[...]
    z = jnp.maximum(_ln(z, g1_ref[...], e1_ref[...]), 0.0)
    z = jnp.dot(z, w2_ref[...], preferred_element_type=jnp.float32) + c2_ref[...]
    z = jnp.maximum(_ln(z, g2_ref[...], e2_ref[...]), 0.0)
    z = jnp.maximum(_ln(z, ng_ref[...], nb_ref[...]), 0.0)
    if residual:
        z = z + h
    o_ref[...] = z
    _quantize(z, qa_ref, qb_ref)


def _gin_pallas(h, agg2a, agg2b, lay, residual):
    nb = _N // _ROWS
    row = pl.BlockSpec((_ROWS, _D), lambda i: (i, 0))
    mat = pl.BlockSpec((_D, _D), lambda i: (0, 0))
    vec = pl.BlockSpec((1, _D), lambda i: (0, 0))
    return pl.pallas_call(
        functools.partial(_gin_body, residual=residual),
        grid=(nb,),
        in_specs=[row, row, row, row, row, mat, vec, vec, vec, mat, vec,
                  vec, vec, vec, vec],
        out_specs=(row, row, row),
        out_shape=(jax.ShapeDtypeStruct((_N, _D), jnp.float32),
                   jax.ShapeDtypeStruct((_N, _D), jnp.int32),
                   jax.ShapeDtypeStruct((_N, _D), jnp.int32)),
    )(h, agg2a[:_N], agg2a[_N:], agg2b[:_N], agg2b[_N:],
      lay["lin1"]["w"], lay["lin1"]["b"].reshape(1, _D),
      lay["ln1g"].reshape(1, _D), lay["ln1b"].reshape(1, _D),
      lay["lin2"]["w"], lay["lin2"]["b"].reshape(1, _D),
      lay["ln2g"].reshape(1, _D), lay["ln2b"].reshape(1, _D),
      lay["ng"].reshape(1, _D), lay["nb"].reshape(1, _D))


def _fuse_body(sa_ref, sb_ref, ta_ref, tb_ref, ca_ref, cb_ref,
               fm_ref, fc_ref, fe_ref,
               wm_ref, bm_ref, gm_ref, em_ref,
               wc_ref, bc_ref, gc_ref, ec_ref,
               we_ref, be_ref, ge_ref, ee_ref,
               f1a_ref, f1b_ref, fb1_ref, fg1_ref, fe1_ref,
               f2a_ref, f2b_ref, fb2_ref, fg2_ref, fe2_ref,
               wv_ref, bv_ref, wo_ref, bo_ref,
               u1a_ref, u1b_ref, ub1_ref, ug1_ref, ue1_ref,
               u2_ref, ub2_ref, ug2_ref, ue2_ref,
               o_ref):
    eps = 1e-5
    cnt = (ca_ref[...][:, :1] + cb_ref[...][:, :1]).astype(jnp.float32)
    sums = ((sa_ref[...].astype(jnp.float32)
             + sb_ref[...].astype(jnp.float32)) * (1.0 / _SA)
            + (ta_ref[...].astype(jnp.float32)
               + tb_ref[...].astype(jnp.float32)) * (1.0 / _SB))
    graph_feat = sums / jnp.maximum(cnt, 1.0)
    em = jnp.dot(fm_ref[...], wm_ref[...], preferred_element_type=jnp.float32) + bm_ref[...]
    em = _gelu(_ln(em, gm_ref[...], em_ref[...]))
    ec = jnp.dot(fc_ref[...], wc_ref[...], preferred_element_type=jnp.float32) + bc_ref[...]
    ec = _gelu(_ln(ec, gc_ref[...], ec_ref[...]))
    ee = jnp.dot(fe_ref[...], we_ref[...], preferred_element_type=jnp.float32) + be_ref[...]
    ee = _gelu(_ln(ee, ge_ref[...], ee_ref[...]))
    c1 = (jnp.dot(em, f1a_ref[...], preferred_element_type=jnp.float32)
          + jnp.dot(ec, f1b_ref[...], preferred_element_type=jnp.float32)
          + fb1_ref[...])
    fused = _gelu(_ln(c1, fg1_ref[...], fe1_ref[...]))
    c2 = (jnp.dot(fused, f2a_ref[...], preferred_element_type=jnp.float32)
          + jnp.dot(ee, f2b_ref[...], preferred_element_type=jnp.float32)
          + fb2_ref[...])
    fp_feat = _gelu(_ln(c2, fg2_ref[...], fe2_ref[...]))
    v = jnp.dot(fp_feat, wv_ref[...], preferred_element_type=jnp.float32) + bv_ref[...]
    fp_attn = jnp.dot(v, wo_ref[...], preferred_element_type=jnp.float32) + bo_ref[...]
    z = (jnp.dot(graph_feat, u1a_ref[...], preferred_element_type=jnp.float32)
         + jnp.dot(fp_attn, u1b_ref[...], preferred_element_type=jnp.float32)
         + ub1_ref[...])
    m = jnp.mean(z, axis=0, keepdims=True)
    va = jnp.mean((z - m) ** 2, axis=0, keepdims=True)
    z = jnp.maximum((z - m) / jnp.sqrt(va + eps) * ug1_ref[...] + ue1_ref[...], 0.0)
    z = jnp.dot(z, u2_ref[...], preferred_element_type=jnp.float32) + ub2_ref[...]
    m = jnp.mean(z, axis=0, keepdims=True)
    va = jnp.mean((z - m) ** 2, axis=0, keepdims=True)
    z = jnp.maximum((z - m) / jnp.sqrt(va + eps) * ug2_ref[...] + ue2_ref[...], 0.0)
    o_ref[...] = z


def _fuse_pallas(sums2a, sums2b, cnts2, fpm, fpc_p, fpe_p, params):
    p = params
    r = lambda a: a.reshape(1, -1)
    fp = p["fp"]
    f1, f2 = p["fuse_fp"]
    at = p["attn"]
    fu = p["fusion"]
    args = (
        sums2a[:_B], sums2a[_B:], sums2b[:_B], sums2b[_B:],
        cnts2[:_B], cnts2[_B:],
        fpm, fpc_p, fpe_p,
        fp["morgan"]["lin"]["w"], r(fp["morgan"]["lin"]["b"]),
        r(fp["morgan"]["g"]), r(fp["morgan"]["b"]),
        _pad_rows(fp["maccs"]["lin"]["w"], 256), r(fp["maccs"]["lin"]["b"]),
        r(fp["maccs"]["g"]), r(fp["maccs"]["b"]),
        _pad_rows(fp["erg"]["lin"]["w"], 512), r(fp["erg"]["lin"]["b"]),
        r(fp["erg"]["g"]), r(fp["erg"]["b"]),
        f1["lin"]["w"][:_H], f1["lin"]["w"][_H:], r(f1["lin"]["b"]),
        r(f1["g"]), r(f1["b"]),
        f2["lin"]["w"][:_H], f2["lin"]["w"][_H:], r(f2["lin"]["b"]),
        r(f2["g"]), r(f2["b"]),
        at["v"]["w"], r(at["v"]["b"]), at["o"]["w"], r(at["o"]["b"]),
        fu["lin1"]["w"][:_H], fu["lin1"]["w"][_H:], r(fu["lin1"]["b"]),
        r(fu["g1"]), r(fu["b1"]),
        fu["lin2"]["w"], r(fu["lin2"]["b"]), r(fu["g2"]), r(fu["b2"]),
    )
    return pl.pallas_call(
        _fuse_body,
        out_shape=jax.ShapeDtypeStruct((_B, _H), jnp.float32),
    )(*args)


def _pad_rows(w, rows):
    return jnp.pad(w, ((0, rows - w.shape[0]), (0, 0)))


def _pad_cols(x, cols):
    return jnp.pad(x, ((0, 0), (0, cols - x.shape[1])))


def kernel(x, fps_morgan, fps_maccs, fps_erg, params, edge_index, graph_ids):
    src_w = edge_index[0].reshape(_NW, _C, _K)
    dst_w = edge_index[1].reshape(_NW, _C, _K)
    zeros_n = jnp.zeros((_N, _D), jnp.int32)

    h, h_fa, h_fb = _ln_pallas(x, params["in_ln_g"], params["in_ln_b"])
    for i, lay in enumerate(params["gin"]):
        agg2a = _seg_sum_sc(h_fa, src_w, dst_w, zeros_n)
        agg2b = _seg_sum_sc(h_fb, src_w, dst_w, zeros_n)
        h, h_fa, h_fb = _gin_pallas(h, agg2a, agg2b, lay, residual=(i > 0))

    pad = _NPAD - _N
    pool_idx = jnp.concatenate(
        [jnp.arange(_N, dtype=jnp.int32),
         jnp.zeros((pad,), jnp.int32)]).reshape(_NW, _CP, _K)
    pool_gid = jnp.concatenate(
        [graph_ids.astype(jnp.int32),
         jnp.full((pad,), _B, jnp.int32)]).reshape(_NW, _CP, _K)
    zeros_p = jnp.zeros((_BPAD, _D), jnp.int32)
    zeros_c = jnp.zeros((_BPAD, _D), jnp.int32)
    ones_k = jnp.ones((_K, _D), jnp.int32)
    sums2a, cnts2 = _pool_sc(h_fa, pool_idx, pool_gid, zeros_p, zeros_c, ones_k)
    sums2b, _ = _pool_sc(h_fb, pool_idx, pool_gid, zeros_p, zeros_c, ones_k)

    fpc_p = _pad_cols(fps_maccs, 256)
    fpe_p = _pad_cols(fps_erg, 512)
    return _fuse_pallas(sums2a, sums2b, cnts2, fps_morgan, fpc_p, fpe_p, params)
